# baseline (speedup 1.0000x reference)
import functools

import jax
import jax.numpy as jnp
from jax import lax
from jax.experimental import pallas as pl
from jax.experimental.pallas import tpu as pltpu
from jax.experimental.pallas import tpu_sc as plsc

N = 10000
NPAD = 10240
E = 320000
D_IN = 128
DH = 64
D_H = 256
DP = 16

NC = 2
NS = 16
NW = NC * NS
C = 125
CHT = E // (NS * C)
CHW = E // (NW * C)
ROWS_PT = NPAD // NS

_MESH = plsc.VectorSubcoreMesh(
    core_axis_name="c", subcore_axis_name="s", num_cores=NC, num_subcores=NS)


@functools.partial(
    pl.kernel,
    out_type=jax.ShapeDtypeStruct((NC, NPAD, DP), jnp.float32),
    mesh=_MESH,
    compiler_params=pltpu.CompilerParams(use_tc_tiling_on_sc=False),
    scratch_types=[
        pltpu.VMEM((CHW, C), jnp.int32),
        pltpu.VMEM((C, DP), jnp.float32),
        pltpu.VMEM_SHARED((NPAD, DP), jnp.float32),
    ] + [pltpu.SemaphoreType.DMA] * 8,
)
def _deg(dsti_hbm, ones_hbm, zero_hbm, out_hbm, di_v, ones_v, acc_sh,
         sem0, sem1, sem2, sem3, sem4, sem5, sem6, sem7):
    c = lax.axis_index("c")
    s = lax.axis_index("s")
    w = c * NS + s
    base = s * ROWS_PT
    pltpu.sync_copy(zero_hbm.at[pl.ds(base, ROWS_PT)],
                    acc_sh.at[pl.ds(base, ROWS_PT)])
    pltpu.sync_copy(dsti_hbm.at[pl.ds(w * CHW, CHW)], di_v)
    pltpu.sync_copy(ones_hbm, ones_v)
    plsc.subcore_barrier()

    sems = (sem0, sem1, sem2, sem3, sem4, sem5, sem6, sem7)

    def body(i, carry):
        g = i * 8
        ds = [pltpu.async_copy(ones_v, acc_sh.at[di_v.at[g + k]],
                               sems[k], add=True)
              for k in range(8)]
        for d in ds:
            d.wait()
        return carry

    lax.fori_loop(0, CHW // 8, body, 0)
    plsc.subcore_barrier()
    pltpu.sync_copy(acc_sh.at[pl.ds(base, ROWS_PT)],
                    out_hbm.at[c, pl.ds(base, ROWS_PT)])


@functools.partial(
    pl.kernel,
    out_type=jax.ShapeDtypeStruct((NC, NPAD, DH), jnp.float32),
    mesh=_MESH,
    compiler_params=pltpu.CompilerParams(use_tc_tiling_on_sc=False),
    scratch_types=[
        pltpu.VMEM((CHT, C), jnp.int32),
        pltpu.VMEM((CHT, C), jnp.int32),
    ] + [pltpu.VMEM((C, DH), jnp.float32)] * 4
      + [pltpu.VMEM_SHARED((NPAD, DH), jnp.float32)]
      + [pltpu.SemaphoreType.DMA] * 8,
)
def _agg128(tab_hbm, srci_hbm, dsti_hbm, zero_hbm, out_hbm,
            si_v, di_v, r0, r1, r2, r3, acc_sh,
            g0, g1, g2, g3, s0, s1, s2, s3):
    c = lax.axis_index("c")
    s = lax.axis_index("s")
    base = s * ROWS_PT
    pltpu.sync_copy(zero_hbm.at[pl.ds(base, ROWS_PT)],
                    acc_sh.at[pl.ds(base, ROWS_PT)])
    pltpu.sync_copy(srci_hbm.at[pl.ds(s * CHT, CHT)], si_v)
    pltpu.sync_copy(dsti_hbm.at[pl.ds(s * CHT, CHT)], di_v)
    plsc.subcore_barrier()

    rows = (r0, r1, r2, r3)
    gsems = (g0, g1, g2, g3)
    ssems = (s0, s1, s2, s3)

    def body(i, carry):
        g = i * 4
        gd = [pltpu.async_copy(tab_hbm.at[c].at[si_v.at[g + k]], rows[k], gsems[k])
              for k in range(4)]
        sd = []
        for k in range(4):
            gd[k].wait()
            sd.append(pltpu.async_copy(rows[k], acc_sh.at[di_v.at[g + k]],
                                       ssems[k], add=True))
        for d in sd:
            d.wait()
        return carry

    lax.fori_loop(0, CHT // 4, body, 0)
    plsc.subcore_barrier()
    pltpu.sync_copy(acc_sh.at[pl.ds(base, ROWS_PT)],
                    out_hbm.at[c, pl.ds(base, ROWS_PT)])


@functools.partial(
    pl.kernel,
    out_type=jax.ShapeDtypeStruct((NC, NPAD, DP), jnp.float32),
    mesh=_MESH,
    compiler_params=pltpu.CompilerParams(use_tc_tiling_on_sc=False),
    scratch_types=[
        pltpu.VMEM((CHW, C), jnp.int32),
        pltpu.VMEM((CHW, C), jnp.int32),
    ] + [pltpu.VMEM((C, DP), jnp.float32)] * 4
      + [pltpu.VMEM_SHARED((NPAD, DP), jnp.float32)]
      + [pltpu.SemaphoreType.DMA] * 8,
)
def _agg16(tab_hbm, srci_hbm, dsti_hbm, zero_hbm, out_hbm,
           si_v, di_v, r0, r1, r2, r3, acc_sh,
           g0, g1, g2, g3, s0, s1, s2, s3):
    c = lax.axis_index("c")
    s = lax.axis_index("s")
    w = c * NS + s
    base = s * ROWS_PT
    pltpu.sync_copy(zero_hbm.at[pl.ds(base, ROWS_PT)],
                    acc_sh.at[pl.ds(base, ROWS_PT)])
    pltpu.sync_copy(srci_hbm.at[pl.ds(w * CHW, CHW)], si_v)
    pltpu.sync_copy(dsti_hbm.at[pl.ds(w * CHW, CHW)], di_v)
    plsc.subcore_barrier()

    rows = (r0, r1, r2, r3)
    gsems = (g0, g1, g2, g3)
    ssems = (s0, s1, s2, s3)

    def body(i, carry):
        g = i * 4
        gd = [pltpu.async_copy(tab_hbm.at[si_v.at[g + k]], rows[k], gsems[k])
              for k in range(4)]
        sd = []
        for k in range(4):
            gd[k].wait()
            sd.append(pltpu.async_copy(rows[k], acc_sh.at[di_v.at[g + k]],
                                       ssems[k], add=True))
        for d in sd:
            d.wait()
        return carry

    lax.fori_loop(0, CHW // 4, body, 0)
    plsc.subcore_barrier()
    pltpu.sync_copy(acc_sh.at[pl.ds(base, ROWS_PT)],
                    out_hbm.at[c, pl.ds(base, ROWS_PT)])


def _tca_body(dp_ref, x_ref, dinv_ref, xs_ref):
    deg = dp_ref[0, :, 0:1] + dp_ref[1, :, 0:1]
    dinv = lax.rsqrt(deg + 1.0)
    dinv_ref[...] = dinv
    v = x_ref[...] * dinv
    xs_ref[0] = v[:, :DH]
    xs_ref[1] = v[:, DH:]


def _tca(degp, xpad):
    return pl.pallas_call(
        _tca_body,
        out_shape=[
            jax.ShapeDtypeStruct((NPAD, 1), jnp.float32),
            jax.ShapeDtypeStruct((NC, NPAD, DH), jnp.float32),
        ],
    )(degp, xpad)


RB = 2560


def _tcu_body(xs_ref, dinv_ref, w1_ref, u_ref):
    dxs = jnp.concatenate([xs_ref[0], xs_ref[1]], axis=1) * dinv_ref[...]
    u_ref[...] = jnp.dot(dxs, w1_ref[...], preferred_element_type=jnp.float32)


def _tcu(xs2, dinv, W1):
    return pl.pallas_call(
        _tcu_body,
        grid=(NPAD // RB,),
        in_specs=[
            pl.BlockSpec((NC, RB, DH), lambda i: (0, i, 0)),
            pl.BlockSpec((RB, 1), lambda i: (i, 0)),
            pl.BlockSpec((D_IN, D_H), lambda i: (0, 0)),
        ],
        out_specs=pl.BlockSpec((RB, D_H), lambda i: (i, 0)),
        out_shape=jax.ShapeDtypeStruct((NPAD, D_H), jnp.float32),
    )(xs2, dinv, W1)


def _tcb_body(s1_ref, u_ref, dinv_ref, w1_ref, b1_ref, w2_ref, ys_ref):
    dinv = dinv_ref[...]
    t = jnp.concatenate([s1_ref[0], s1_ref[1]], axis=1) * dinv
    h = jnp.maximum(
        jnp.dot(t, w1_ref[...], preferred_element_type=jnp.float32)
        + u_ref[...] + b1_ref[...],
        0.0)
    y = jnp.dot(h, w2_ref[...], preferred_element_type=jnp.float32)
    ys_ref[...] = y * dinv


def _tcb(s1p, u, dinv, W1, b1, W2p):
    return pl.pallas_call(
        _tcb_body,
        grid=(NPAD // RB,),
        in_specs=[
            pl.BlockSpec((NC, RB, DH), lambda i: (0, i, 0)),
            pl.BlockSpec((RB, D_H), lambda i: (i, 0)),
            pl.BlockSpec((RB, 1), lambda i: (i, 0)),
            pl.BlockSpec((D_IN, D_H), lambda i: (0, 0)),
            pl.BlockSpec((1, D_H), lambda i: (0, 0)),
            pl.BlockSpec((D_H, DP), lambda i: (0, 0)),
        ],
        out_specs=pl.BlockSpec((RB, DP), lambda i: (i, 0)),
        out_shape=jax.ShapeDtypeStruct((NPAD, DP), jnp.float32),
    )(s1p, u, dinv, W1, b1, W2p)


def _tcc_body(s2p_ref, ys_ref, dinv_ref, b2_ref, out_ref):
    out_ref[...] = (s2p_ref[0] + s2p_ref[1] + ys_ref[...]) * dinv_ref[...] + b2_ref[...]


def _tcc(s2p, ys, dinv, b2p):
    return pl.pallas_call(
        _tcc_body,
        out_shape=jax.ShapeDtypeStruct((NPAD, DP), jnp.float32),
    )(s2p, ys, dinv, b2p)


def kernel(x, edge_index, W1, b1, W2, b2):
    src2 = edge_index[0].reshape(NS * CHT, C)
    dst2 = edge_index[1].reshape(NS * CHT, C)
    xpad = jnp.pad(x, ((0, NPAD - N), (0, 0)))
    z64 = jnp.zeros((NPAD, DH), jnp.float32)
    z16 = jnp.zeros((NPAD, DP), jnp.float32)
    ones16 = jnp.ones((C, DP), jnp.float32)
    W2p = jnp.pad(W2, ((0, 0), (0, DP - W2.shape[1])))
    b1r = b1.reshape(1, D_H)
    b2p = jnp.pad(b2, (0, DP - b2.shape[0])).reshape(1, DP)

    degp = _deg(dst2, ones16, z16)
    dinv, xs2 = _tca(degp, xpad)
    s1p = _agg128(xs2, src2, dst2, z64)
    u = _tcu(xs2, dinv, W1)
    ys = _tcb(s1p, u, dinv, W1, b1r, W2p)
    s2p = _agg16(ys, src2, dst2, z16)
    out16 = _tcc(s2p, ys, dinv, b2p)
    return out16[:N, :2]


# speedup vs baseline: 1.0303x; 1.0303x over previous
import functools

import jax
import jax.numpy as jnp
from jax import lax
from jax.experimental import pallas as pl
from jax.experimental.pallas import tpu as pltpu
from jax.experimental.pallas import tpu_sc as plsc

N = 10000
NPAD = 10240
E = 320000
D_IN = 128
DH = 64
D_H = 256
DP = 16

NC = 2
NS = 16
NW = NC * NS
C = 125
CHT = E // (NS * C)
CHW = E // (NW * C)
ROWS_PT = NPAD // NS

_MESH = plsc.VectorSubcoreMesh(
    core_axis_name="c", subcore_axis_name="s", num_cores=NC, num_subcores=NS)


@functools.partial(
    pl.kernel,
    out_type=jax.ShapeDtypeStruct((NC, NPAD, DP), jnp.float32),
    mesh=_MESH,
    compiler_params=pltpu.CompilerParams(use_tc_tiling_on_sc=False),
    scratch_types=[
        pltpu.VMEM((CHW, C), jnp.int32),
        pltpu.VMEM((C, DP), jnp.float32),
        pltpu.VMEM_SHARED((NPAD, DP), jnp.float32),
    ] + [pltpu.SemaphoreType.DMA] * 8,
)
def _deg(ei_hbm, ones_hbm, zero_hbm, out_hbm, di_v, ones_v, acc_sh,
         sem0, sem1, sem2, sem3, sem4, sem5, sem6, sem7):
    c = lax.axis_index("c")
    s = lax.axis_index("s")
    w = c * NS + s
    base = s * ROWS_PT
    pltpu.sync_copy(zero_hbm.at[pl.ds(base, ROWS_PT)],
                    acc_sh.at[pl.ds(base, ROWS_PT)])
    pltpu.sync_copy(ei_hbm.at[1, pl.ds(w * CHW, CHW)], di_v)
    pltpu.sync_copy(ones_hbm, ones_v)
    plsc.subcore_barrier()

    sems = (sem0, sem1, sem2, sem3, sem4, sem5, sem6, sem7)

    def body(i, carry):
        g = i * 8
        ds = [pltpu.async_copy(ones_v, acc_sh.at[di_v.at[g + k]],
                               sems[k], add=True)
              for k in range(8)]
        for d in ds:
            d.wait()
        return carry

    lax.fori_loop(0, CHW // 8, body, 0)
    plsc.subcore_barrier()
    pltpu.sync_copy(acc_sh.at[pl.ds(base, ROWS_PT)],
                    out_hbm.at[c, pl.ds(base, ROWS_PT)])


@functools.partial(
    pl.kernel,
    out_type=jax.ShapeDtypeStruct((NC, NPAD, DH), jnp.float32),
    mesh=_MESH,
    compiler_params=pltpu.CompilerParams(use_tc_tiling_on_sc=False),
    scratch_types=[
        pltpu.VMEM((CHT, C), jnp.int32),
        pltpu.VMEM((CHT, C), jnp.int32),
    ] + [pltpu.VMEM((C, DH), jnp.float32)] * 4
      + [pltpu.VMEM_SHARED((NPAD, DH), jnp.float32)]
      + [pltpu.SemaphoreType.DMA] * 8,
)
def _agg128(tab_hbm, ei_hbm, zero_hbm, out_hbm,
            si_v, di_v, r0, r1, r2, r3, acc_sh,
            g0, g1, g2, g3, s0, s1, s2, s3):
    c = lax.axis_index("c")
    s = lax.axis_index("s")
    base = s * ROWS_PT
    pltpu.sync_copy(zero_hbm.at[pl.ds(base, ROWS_PT)],
                    acc_sh.at[pl.ds(base, ROWS_PT)])
    pltpu.sync_copy(ei_hbm.at[0, pl.ds(s * CHT, CHT)], si_v)
    pltpu.sync_copy(ei_hbm.at[1, pl.ds(s * CHT, CHT)], di_v)
    plsc.subcore_barrier()

    rows = (r0, r1, r2, r3)
    gsems = (g0, g1, g2, g3)
    ssems = (s0, s1, s2, s3)

    def body(i, carry):
        g = i * 4
        gd = [pltpu.async_copy(tab_hbm.at[c].at[si_v.at[g + k]], rows[k],
                               gsems[k])
              for k in range(4)]
        sd = []
        for k in range(4):
            gd[k].wait()
            sd.append(pltpu.async_copy(rows[k], acc_sh.at[di_v.at[g + k]],
                                       ssems[k], add=True))
        for d in sd:
            d.wait()
        return carry

    lax.fori_loop(0, CHT // 4, body, 0)
    plsc.subcore_barrier()
    pltpu.sync_copy(acc_sh.at[pl.ds(base, ROWS_PT)],
                    out_hbm.at[c, pl.ds(base, ROWS_PT)])


@functools.partial(
    pl.kernel,
    out_type=jax.ShapeDtypeStruct((NC, NPAD, DP), jnp.float32),
    mesh=_MESH,
    compiler_params=pltpu.CompilerParams(use_tc_tiling_on_sc=False),
    scratch_types=[
        pltpu.VMEM((CHW, C), jnp.int32),
        pltpu.VMEM((CHW, C), jnp.int32),
    ] + [pltpu.VMEM((C, DP), jnp.float32)] * 4
      + [pltpu.VMEM_SHARED((NPAD, DP), jnp.float32)]
      + [pltpu.SemaphoreType.DMA] * 8,
)
def _agg16(tab_hbm, ei_hbm, zero_hbm, out_hbm,
           si_v, di_v, r0, r1, r2, r3, acc_sh,
           g0, g1, g2, g3, s0, s1, s2, s3):
    c = lax.axis_index("c")
    s = lax.axis_index("s")
    w = c * NS + s
    base = s * ROWS_PT
    pltpu.sync_copy(zero_hbm.at[pl.ds(base, ROWS_PT)],
                    acc_sh.at[pl.ds(base, ROWS_PT)])
    pltpu.sync_copy(ei_hbm.at[0, pl.ds(w * CHW, CHW)], si_v)
    pltpu.sync_copy(ei_hbm.at[1, pl.ds(w * CHW, CHW)], di_v)
    plsc.subcore_barrier()

    rows = (r0, r1, r2, r3)
    gsems = (g0, g1, g2, g3)
    ssems = (s0, s1, s2, s3)

    def body(i, carry):
        g = i * 4
        gd = [pltpu.async_copy(tab_hbm.at[si_v.at[g + k]], rows[k], gsems[k])
              for k in range(4)]
        sd = []
        for k in range(4):
            gd[k].wait()
            sd.append(pltpu.async_copy(rows[k], acc_sh.at[di_v.at[g + k]],
                                       ssems[k], add=True))
        for d in sd:
            d.wait()
        return carry

    lax.fori_loop(0, CHW // 4, body, 0)
    plsc.subcore_barrier()
    pltpu.sync_copy(acc_sh.at[pl.ds(base, ROWS_PT)],
                    out_hbm.at[c, pl.ds(base, ROWS_PT)])


def _tca_body(dp_ref, x_ref, dinv_ref, xs_ref):
    deg = dp_ref[0, :, 0:1] + dp_ref[1, :, 0:1]
    dinv = lax.rsqrt(deg + 1.0)
    dinv_ref[...] = dinv
    v = x_ref[...] * dinv
    xs_ref[0] = v[:, :DH]
    xs_ref[1] = v[:, DH:]


def _tca(degp, xpad):
    return pl.pallas_call(
        _tca_body,
        out_shape=[
            jax.ShapeDtypeStruct((NPAD, 1), jnp.float32),
            jax.ShapeDtypeStruct((NC, NPAD, DH), jnp.float32),
        ],
    )(degp, xpad)


RB = 2560


def _tcu_body(xs_ref, dinv_ref, w1_ref, u_ref):
    dxs = jnp.concatenate([xs_ref[0], xs_ref[1]], axis=1) * dinv_ref[...]
    u_ref[...] = jnp.dot(dxs, w1_ref[...], preferred_element_type=jnp.float32)


def _tcu(xs2, dinv, W1):
    return pl.pallas_call(
        _tcu_body,
        grid=(NPAD // RB,),
        in_specs=[
            pl.BlockSpec((NC, RB, DH), lambda i: (0, i, 0)),
            pl.BlockSpec((RB, 1), lambda i: (i, 0)),
            pl.BlockSpec((D_IN, D_H), lambda i: (0, 0)),
        ],
        out_specs=pl.BlockSpec((RB, D_H), lambda i: (i, 0)),
        out_shape=jax.ShapeDtypeStruct((NPAD, D_H), jnp.float32),
    )(xs2, dinv, W1)


def _tcb_body(s1_ref, u_ref, dinv_ref, w1_ref, b1_ref, w2_ref, ys_ref):
    dinv = dinv_ref[...]
    t = jnp.concatenate([s1_ref[0], s1_ref[1]], axis=1) * dinv
    h = jnp.maximum(
        jnp.dot(t, w1_ref[...], preferred_element_type=jnp.float32)
        + u_ref[...] + b1_ref[...],
        0.0)
    y = jnp.dot(h, w2_ref[...], preferred_element_type=jnp.float32)
    ys_ref[...] = y * dinv


def _tcb(s1p, u, dinv, W1, b1, W2p):
    return pl.pallas_call(
        _tcb_body,
        grid=(NPAD // RB,),
        in_specs=[
            pl.BlockSpec((NC, RB, DH), lambda i: (0, i, 0)),
            pl.BlockSpec((RB, D_H), lambda i: (i, 0)),
            pl.BlockSpec((RB, 1), lambda i: (i, 0)),
            pl.BlockSpec((D_IN, D_H), lambda i: (0, 0)),
            pl.BlockSpec((1, D_H), lambda i: (0, 0)),
            pl.BlockSpec((D_H, DP), lambda i: (0, 0)),
        ],
        out_specs=pl.BlockSpec((RB, DP), lambda i: (i, 0)),
        out_shape=jax.ShapeDtypeStruct((NPAD, DP), jnp.float32),
    )(s1p, u, dinv, W1, b1, W2p)


def _tcc_body(s2p_ref, ys_ref, dinv_ref, b2_ref, out_ref):
    out_ref[...] = ((s2p_ref[0] + s2p_ref[1] + ys_ref[...]) * dinv_ref[...]
                    + b2_ref[...])


def _tcc(s2p, ys, dinv, b2p):
    return pl.pallas_call(
        _tcc_body,
        out_shape=jax.ShapeDtypeStruct((NPAD, DP), jnp.float32),
    )(s2p, ys, dinv, b2p)


def kernel(x, edge_index, W1, b1, W2, b2):
    ei3 = edge_index.reshape(2, NS * CHT, C)
    xpad = jnp.pad(x, ((0, NPAD - N), (0, 0)))
    z64 = jnp.zeros((NPAD, DH), jnp.float32)
    z16 = jnp.zeros((NPAD, DP), jnp.float32)
    ones16 = jnp.ones((C, DP), jnp.float32)
    W2p = jnp.pad(W2, ((0, 0), (0, DP - W2.shape[1])))
    b1r = b1.reshape(1, D_H)
    b2p = jnp.pad(b2, (0, DP - b2.shape[0])).reshape(1, DP)

    degp = _deg(ei3, ones16, z16)
    dinv, xs2 = _tca(degp, xpad)
    s1p = _agg128(xs2, ei3, z64)
    u = _tcu(xs2, dinv, W1)
    ys = _tcb(s1p, u, dinv, W1, b1r, W2p)
    s2p = _agg16(ys, ei3, z16)
    out16 = _tcc(s2p, ys, dinv, b2p)
    return out16[:N, :2]


# speedup vs baseline: 1.0347x; 1.0043x over previous
import functools

import jax
import jax.numpy as jnp
from jax import lax
from jax.experimental import pallas as pl
from jax.experimental.pallas import tpu as pltpu
from jax.experimental.pallas import tpu_sc as plsc

N = 10000
NPAD = 10240
E = 320000
D_IN = 128
DH = 64
D_H = 256
DP = 16

NC = 2
NS = 16
NW = NC * NS
C = 125
CHT = E // (NS * C)
CHW = E // (NW * C)
ROWS_PT = NPAD // NS

_MESH = plsc.VectorSubcoreMesh(
    core_axis_name="c", subcore_axis_name="s", num_cores=NC, num_subcores=NS)


@functools.partial(
    pl.kernel,
    out_type=jax.ShapeDtypeStruct((NC, NPAD, DP), jnp.float32),
    mesh=_MESH,
    compiler_params=pltpu.CompilerParams(use_tc_tiling_on_sc=False),
    scratch_types=[
        pltpu.VMEM((CHW, C), jnp.int32),
        pltpu.VMEM((C, DP), jnp.float32),
        pltpu.VMEM_SHARED((NPAD, DP), jnp.float32),
    ] + [pltpu.SemaphoreType.DMA] * 8,
)
def _deg(ei_hbm, ones_hbm, zero_hbm, out_hbm, di_v, ones_v, acc_sh,
         sem0, sem1, sem2, sem3, sem4, sem5, sem6, sem7):
    c = lax.axis_index("c")
    s = lax.axis_index("s")
    w = c * NS + s
    base = s * ROWS_PT
    pltpu.sync_copy(zero_hbm.at[pl.ds(base, ROWS_PT)],
                    acc_sh.at[pl.ds(base, ROWS_PT)])
    pltpu.sync_copy(ei_hbm.at[1, pl.ds(w * CHW, CHW)], di_v)
    pltpu.sync_copy(ones_hbm, ones_v)
    plsc.subcore_barrier()

    sems = (sem0, sem1, sem2, sem3, sem4, sem5, sem6, sem7)

    def body(i, carry):
        g = i * 8
        ds = [pltpu.async_copy(ones_v, acc_sh.at[di_v.at[g + k]],
                               sems[k], add=True)
              for k in range(8)]
        for d in ds:
            d.wait()
        return carry

    lax.fori_loop(0, CHW // 8, body, 0)
    plsc.subcore_barrier()
    pltpu.sync_copy(acc_sh.at[pl.ds(base, ROWS_PT)],
                    out_hbm.at[c, pl.ds(base, ROWS_PT)])


@functools.partial(
    pl.kernel,
    out_type=jax.ShapeDtypeStruct((NC, NPAD, DH), jnp.float32),
    mesh=_MESH,
    compiler_params=pltpu.CompilerParams(use_tc_tiling_on_sc=False),
    scratch_types=[
        pltpu.VMEM((CHT, C), jnp.int32),
        pltpu.VMEM((CHT, C), jnp.int32),
    ] + [pltpu.VMEM((C, DH), jnp.float32)] * 4
      + [pltpu.VMEM_SHARED((NPAD, DH), jnp.float32)]
      + [pltpu.SemaphoreType.DMA] * 8,
)
def _agg128(tab_hbm, ei_hbm, zero_hbm, out_hbm,
            si_v, di_v, r0, r1, r2, r3, acc_sh,
            g0, g1, g2, g3, s0, s1, s2, s3):
    c = lax.axis_index("c")
    s = lax.axis_index("s")
    base = s * ROWS_PT
    pltpu.sync_copy(zero_hbm.at[pl.ds(base, ROWS_PT)],
                    acc_sh.at[pl.ds(base, ROWS_PT)])
    pltpu.sync_copy(ei_hbm.at[0, pl.ds(s * CHT, CHT)], si_v)
    pltpu.sync_copy(ei_hbm.at[1, pl.ds(s * CHT, CHT)], di_v)
    plsc.subcore_barrier()

    rows = (r0, r1, r2, r3)
    gsems = (g0, g1, g2, g3)
    ssems = (s0, s1, s2, s3)

    def body(i, carry):
        g = i * 4
        gd = [pltpu.async_copy(tab_hbm.at[c].at[si_v.at[g + k]], rows[k],
                               gsems[k])
              for k in range(4)]
        sd = []
        for k in range(4):
            gd[k].wait()
            sd.append(pltpu.async_copy(rows[k], acc_sh.at[di_v.at[g + k]],
                                       ssems[k], add=True))
        for d in sd:
            d.wait()
        return carry

    lax.fori_loop(0, CHT // 4, body, 0)
    plsc.subcore_barrier()
    pltpu.sync_copy(acc_sh.at[pl.ds(base, ROWS_PT)],
                    out_hbm.at[c, pl.ds(base, ROWS_PT)])


@functools.partial(
    pl.kernel,
    out_type=jax.ShapeDtypeStruct((NC, NPAD, DP), jnp.float32),
    mesh=_MESH,
    compiler_params=pltpu.CompilerParams(use_tc_tiling_on_sc=False),
    scratch_types=[
        pltpu.VMEM((CHW, C), jnp.int32),
        pltpu.VMEM((CHW, C), jnp.int32),
    ] + [pltpu.VMEM((C, DP), jnp.float32)] * 4
      + [pltpu.VMEM_SHARED((NPAD, DP), jnp.float32)]
      + [pltpu.SemaphoreType.DMA] * 8,
)
def _agg16(tab_hbm, ei_hbm, zero_hbm, out_hbm,
           si_v, di_v, r0, r1, r2, r3, acc_sh,
           g0, g1, g2, g3, s0, s1, s2, s3):
    c = lax.axis_index("c")
    s = lax.axis_index("s")
    w = c * NS + s
    base = s * ROWS_PT
    pltpu.sync_copy(zero_hbm.at[pl.ds(base, ROWS_PT)],
                    acc_sh.at[pl.ds(base, ROWS_PT)])
    pltpu.sync_copy(ei_hbm.at[0, pl.ds(w * CHW, CHW)], si_v)
    pltpu.sync_copy(ei_hbm.at[1, pl.ds(w * CHW, CHW)], di_v)
    plsc.subcore_barrier()

    rows = (r0, r1, r2, r3)
    gsems = (g0, g1, g2, g3)
    ssems = (s0, s1, s2, s3)

    def body(i, carry):
        g = i * 4
        gd = [pltpu.async_copy(tab_hbm.at[si_v.at[g + k]], rows[k], gsems[k])
              for k in range(4)]
        sd = []
        for k in range(4):
            gd[k].wait()
            sd.append(pltpu.async_copy(rows[k], acc_sh.at[di_v.at[g + k]],
                                       ssems[k], add=True))
        for d in sd:
            d.wait()
        return carry

    lax.fori_loop(0, CHW // 4, body, 0)
    plsc.subcore_barrier()
    pltpu.sync_copy(acc_sh.at[pl.ds(base, ROWS_PT)],
                    out_hbm.at[c, pl.ds(base, ROWS_PT)])


def _tca_body(dp_ref, x_ref, dinv_ref, xs_ref):
    deg = dp_ref[0, :, 0:1] + dp_ref[1, :, 0:1]
    dinv = lax.rsqrt(deg + 1.0)
    dinv_ref[...] = dinv
    v = x_ref[...] * dinv
    xs_ref[0] = v[:, :DH]
    xs_ref[1] = v[:, DH:]


def _tca(degp, xpad):
    return pl.pallas_call(
        _tca_body,
        out_shape=[
            jax.ShapeDtypeStruct((NPAD, 1), jnp.float32),
            jax.ShapeDtypeStruct((NC, NPAD, DH), jnp.float32),
        ],
    )(degp, xpad)


RB = 2560


def _tcu_body(xs_ref, dinv_ref, w1_ref, u_ref):
    dxs = jnp.concatenate([xs_ref[0], xs_ref[1]], axis=1) * dinv_ref[...]
    u_ref[...] = jnp.dot(dxs, w1_ref[...], preferred_element_type=jnp.float32)


def _tcu(xs2, dinv, W1):
    return pl.pallas_call(
        _tcu_body,
        grid=(NPAD // RB,),
        in_specs=[
            pl.BlockSpec((NC, RB, DH), lambda i: (0, i, 0)),
            pl.BlockSpec((RB, 1), lambda i: (i, 0)),
            pl.BlockSpec((D_IN, D_H), lambda i: (0, 0)),
        ],
        out_specs=pl.BlockSpec((RB, D_H), lambda i: (i, 0)),
        out_shape=jax.ShapeDtypeStruct((NPAD, D_H), jnp.float32),
    )(xs2, dinv, W1)


def _tcb_body(s1_ref, u_ref, dinv_ref, w1_ref, b1_ref, w2_ref, ys_ref):
    dinv = dinv_ref[...]
    t = jnp.concatenate([s1_ref[0], s1_ref[1]], axis=1) * dinv
    h = jnp.maximum(
        jnp.dot(t, w1_ref[...], preferred_element_type=jnp.float32)
        + u_ref[...] + b1_ref[...],
        0.0)
    y = jnp.dot(h, w2_ref[...], preferred_element_type=jnp.float32)
    ys_ref[...] = y * dinv


def _tcb(s1p, u, dinv, W1, b1, W2p):
    return pl.pallas_call(
        _tcb_body,
        grid=(NPAD // RB,),
        in_specs=[
            pl.BlockSpec((NC, RB, DH), lambda i: (0, i, 0)),
            pl.BlockSpec((RB, D_H), lambda i: (i, 0)),
            pl.BlockSpec((RB, 1), lambda i: (i, 0)),
            pl.BlockSpec((D_IN, D_H), lambda i: (0, 0)),
            pl.BlockSpec((1, D_H), lambda i: (0, 0)),
            pl.BlockSpec((D_H, DP), lambda i: (0, 0)),
        ],
        out_specs=pl.BlockSpec((RB, DP), lambda i: (i, 0)),
        out_shape=jax.ShapeDtypeStruct((NPAD, DP), jnp.float32),
    )(s1p, u, dinv, W1, b1, W2p)


def _tcc_body(s2p_ref, ys_ref, dinv_ref, b2_ref, out_ref):
    v = ((s2p_ref[0] + s2p_ref[1] + ys_ref[...]) * dinv_ref[...]
         + b2_ref[...])
    out_ref[...] = v[:N, 0:2]


def _tcc(s2p, ys, dinv, b2p):
    return pl.pallas_call(
        _tcc_body,
        out_shape=jax.ShapeDtypeStruct((N, 2), jnp.float32),
    )(s2p, ys, dinv, b2p)


def kernel(x, edge_index, W1, b1, W2, b2):
    ei3 = edge_index.reshape(2, NS * CHT, C)
    xpad = jnp.pad(x, ((0, NPAD - N), (0, 0)))
    z64 = jnp.zeros((NPAD, DH), jnp.float32)
    z16 = jnp.zeros((NPAD, DP), jnp.float32)
    ones16 = jnp.ones((C, DP), jnp.float32)
    W2p = jnp.pad(W2, ((0, 0), (0, DP - W2.shape[1])))
    b1r = b1.reshape(1, D_H)
    b2p = jnp.pad(b2, (0, DP - b2.shape[0])).reshape(1, DP)

    degp = _deg(ei3, ones16, z16)
    dinv, xs2 = _tca(degp, xpad)
    s1p = _agg128(xs2, ei3, z64)
    u = _tcu(xs2, dinv, W1)
    ys = _tcb(s1p, u, dinv, W1, b1r, W2p)
    s2p = _agg16(ys, ei3, z16)
    return _tcc(s2p, ys, dinv, b2p)


# speedup vs baseline: 1.0609x; 1.0253x over previous
import functools

import jax
import jax.numpy as jnp
from jax import lax
from jax.experimental import pallas as pl
from jax.experimental.pallas import tpu as pltpu
from jax.experimental.pallas import tpu_sc as plsc

N = 10000
NPAD = 10240
E = 320000
D_IN = 128
DH = 64
D_H = 256
DP = 16

NC = 2
NS = 16
NW = NC * NS
C = 125
CHT = E // (NS * C)
CHW = E // (NW * C)
ROWS_PT = NPAD // NS

_MESH = plsc.VectorSubcoreMesh(
    core_axis_name="c", subcore_axis_name="s", num_cores=NC, num_subcores=NS)


@functools.partial(
    pl.kernel,
    out_type=jax.ShapeDtypeStruct((NC, NPAD, DP), jnp.float32),
    mesh=_MESH,
    compiler_params=pltpu.CompilerParams(use_tc_tiling_on_sc=False),
    scratch_types=[
        pltpu.VMEM((CHW, C), jnp.int32),
        pltpu.VMEM((C, DP), jnp.float32),
        pltpu.VMEM_SHARED((NPAD, DP), jnp.float32),
    ] + [pltpu.SemaphoreType.DMA] * 8,
)
def _deg(ei_hbm, ones_hbm, zero_hbm, out_hbm, di_v, ones_v, acc_sh,
         sem0, sem1, sem2, sem3, sem4, sem5, sem6, sem7):
    c = lax.axis_index("c")
    s = lax.axis_index("s")
    w = c * NS + s
    base = s * ROWS_PT
    pltpu.sync_copy(zero_hbm.at[pl.ds(base, ROWS_PT)],
                    acc_sh.at[pl.ds(base, ROWS_PT)])
    pltpu.sync_copy(ei_hbm.at[1, pl.ds(w * CHW, CHW)], di_v)
    pltpu.sync_copy(ones_hbm, ones_v)
    plsc.subcore_barrier()

    sems = (sem0, sem1, sem2, sem3, sem4, sem5, sem6, sem7)

    def body(i, carry):
        g = i * 8
        ds = [pltpu.async_copy(ones_v, acc_sh.at[di_v.at[g + k]],
                               sems[k], add=True)
              for k in range(8)]
        for d in ds:
            d.wait()
        return carry

    lax.fori_loop(0, CHW // 8, body, 0)
    plsc.subcore_barrier()
    pltpu.sync_copy(acc_sh.at[pl.ds(base, ROWS_PT)],
                    out_hbm.at[c, pl.ds(base, ROWS_PT)])


@functools.partial(
    pl.kernel,
    out_type=jax.ShapeDtypeStruct((NC, NPAD, DH), jnp.float32),
    mesh=_MESH,
    compiler_params=pltpu.CompilerParams(use_tc_tiling_on_sc=False),
    scratch_types=[
        pltpu.VMEM((CHT, C), jnp.int32),
        pltpu.VMEM((CHT, C), jnp.int32),
    ] + [pltpu.VMEM((C, DH), jnp.float32)] * 4
      + [pltpu.VMEM_SHARED((NPAD, DH), jnp.float32)]
      + [pltpu.SemaphoreType.DMA] * 8,
)
def _agg128(tab_hbm, ei_hbm, zero_hbm, out_hbm,
            si_v, di_v, r0, r1, r2, r3, acc_sh,
            g0, g1, g2, g3, s0, s1, s2, s3):
    c = lax.axis_index("c")
    s = lax.axis_index("s")
    base = s * ROWS_PT
    pltpu.sync_copy(zero_hbm.at[pl.ds(base, ROWS_PT)],
                    acc_sh.at[pl.ds(base, ROWS_PT)])
    pltpu.sync_copy(ei_hbm.at[0, pl.ds(s * CHT, CHT)], si_v)
    pltpu.sync_copy(ei_hbm.at[1, pl.ds(s * CHT, CHT)], di_v)
    plsc.subcore_barrier()

    rows = (r0, r1, r2, r3)
    gsems = (g0, g1, g2, g3)
    ssems = (s0, s1, s2, s3)

    def body(i, carry):
        g = i * 4
        gd = [pltpu.async_copy(tab_hbm.at[c].at[si_v.at[g + k]], rows[k],
                               gsems[k])
              for k in range(4)]
        sd = []
        for k in range(4):
            gd[k].wait()
            sd.append(pltpu.async_copy(rows[k], acc_sh.at[di_v.at[g + k]],
                                       ssems[k], add=True))
        for d in sd:
            d.wait()
        return carry

    lax.fori_loop(0, CHT // 4, body, 0)
    plsc.subcore_barrier()
    pltpu.sync_copy(acc_sh.at[pl.ds(base, ROWS_PT)],
                    out_hbm.at[c, pl.ds(base, ROWS_PT)])


@functools.partial(
    pl.kernel,
    out_type=jax.ShapeDtypeStruct((NC, NPAD, DP), jnp.float32),
    mesh=_MESH,
    compiler_params=pltpu.CompilerParams(use_tc_tiling_on_sc=False),
    scratch_types=[
        pltpu.VMEM((CHW, C), jnp.int32),
        pltpu.VMEM((CHW, C), jnp.int32),
    ] + [pltpu.VMEM((C, DP), jnp.float32)] * 8
      + [pltpu.VMEM_SHARED((NPAD, DP), jnp.float32)]
      + [pltpu.SemaphoreType.DMA] * 16,
)
def _agg16(tab_hbm, ei_hbm, zero_hbm, out_hbm,
           si_v, di_v, r0, r1, r2, r3, r4, r5, r6, r7, acc_sh,
           g0, g1, g2, g3, g4, g5, g6, g7,
           s0, s1, s2, s3, s4, s5, s6, s7):
    c = lax.axis_index("c")
    s = lax.axis_index("s")
    w = c * NS + s
    base = s * ROWS_PT
    pltpu.sync_copy(zero_hbm.at[pl.ds(base, ROWS_PT)],
                    acc_sh.at[pl.ds(base, ROWS_PT)])
    pltpu.sync_copy(ei_hbm.at[0, pl.ds(w * CHW, CHW)], si_v)
    pltpu.sync_copy(ei_hbm.at[1, pl.ds(w * CHW, CHW)], di_v)
    plsc.subcore_barrier()

    rows = (r0, r1, r2, r3, r4, r5, r6, r7)
    gsems = (g0, g1, g2, g3, g4, g5, g6, g7)
    ssems = (s0, s1, s2, s3, s4, s5, s6, s7)

    def body(i, carry):
        g = i * 8
        gd = [pltpu.async_copy(tab_hbm.at[si_v.at[g + k]], rows[k], gsems[k])
              for k in range(8)]
        sd = []
        for k in range(8):
            gd[k].wait()
            sd.append(pltpu.async_copy(rows[k], acc_sh.at[di_v.at[g + k]],
                                       ssems[k], add=True))
        for d in sd:
            d.wait()
        return carry

    lax.fori_loop(0, CHW // 8, body, 0)
    plsc.subcore_barrier()
    pltpu.sync_copy(acc_sh.at[pl.ds(base, ROWS_PT)],
                    out_hbm.at[c, pl.ds(base, ROWS_PT)])


def _tca_body(dp_ref, x_ref, dinv_ref, xs_ref):
    deg = dp_ref[0, :, 0:1] + dp_ref[1, :, 0:1]
    dinv = lax.rsqrt(deg + 1.0)
    dinv_ref[...] = dinv
    v = x_ref[...] * dinv
    xs_ref[0] = v[:, :DH]
    xs_ref[1] = v[:, DH:]


def _tca(degp, xpad):
    return pl.pallas_call(
        _tca_body,
        out_shape=[
            jax.ShapeDtypeStruct((NPAD, 1), jnp.float32),
            jax.ShapeDtypeStruct((NC, NPAD, DH), jnp.float32),
        ],
    )(degp, xpad)


RB = 2560


def _tcu_body(xs_ref, dinv_ref, w1_ref, u_ref):
    dxs = jnp.concatenate([xs_ref[0], xs_ref[1]], axis=1) * dinv_ref[...]
    u_ref[...] = jnp.dot(dxs, w1_ref[...], preferred_element_type=jnp.float32)


def _tcu(xs2, dinv, W1):
    return pl.pallas_call(
        _tcu_body,
        grid=(NPAD // RB,),
        in_specs=[
            pl.BlockSpec((NC, RB, DH), lambda i: (0, i, 0)),
            pl.BlockSpec((RB, 1), lambda i: (i, 0)),
            pl.BlockSpec((D_IN, D_H), lambda i: (0, 0)),
        ],
        out_specs=pl.BlockSpec((RB, D_H), lambda i: (i, 0)),
        out_shape=jax.ShapeDtypeStruct((NPAD, D_H), jnp.float32),
    )(xs2, dinv, W1)


def _tcb_body(s1_ref, u_ref, dinv_ref, w1_ref, b1_ref, w2_ref, ys_ref):
    dinv = dinv_ref[...]
    t = jnp.concatenate([s1_ref[0], s1_ref[1]], axis=1) * dinv
    h = jnp.maximum(
        jnp.dot(t, w1_ref[...], preferred_element_type=jnp.float32)
        + u_ref[...] + b1_ref[...],
        0.0)
    y = jnp.dot(h, w2_ref[...], preferred_element_type=jnp.float32)
    ys_ref[...] = y * dinv


def _tcb(s1p, u, dinv, W1, b1, W2p):
    return pl.pallas_call(
        _tcb_body,
        grid=(NPAD // RB,),
        in_specs=[
            pl.BlockSpec((NC, RB, DH), lambda i: (0, i, 0)),
            pl.BlockSpec((RB, D_H), lambda i: (i, 0)),
            pl.BlockSpec((RB, 1), lambda i: (i, 0)),
            pl.BlockSpec((D_IN, D_H), lambda i: (0, 0)),
            pl.BlockSpec((1, D_H), lambda i: (0, 0)),
            pl.BlockSpec((D_H, DP), lambda i: (0, 0)),
        ],
        out_specs=pl.BlockSpec((RB, DP), lambda i: (i, 0)),
        out_shape=jax.ShapeDtypeStruct((NPAD, DP), jnp.float32),
    )(s1p, u, dinv, W1, b1, W2p)


def _tcc_body(s2p_ref, ys_ref, dinv_ref, b2_ref, out_ref):
    v = ((s2p_ref[0] + s2p_ref[1] + ys_ref[...]) * dinv_ref[...]
         + b2_ref[...])
    out_ref[...] = v[:N, 0:2]


def _tcc(s2p, ys, dinv, b2p):
    return pl.pallas_call(
        _tcc_body,
        out_shape=jax.ShapeDtypeStruct((N, 2), jnp.float32),
    )(s2p, ys, dinv, b2p)


def kernel(x, edge_index, W1, b1, W2, b2):
    ei3 = edge_index.reshape(2, NS * CHT, C)
    xpad = jnp.pad(x, ((0, NPAD - N), (0, 0)))
    z64 = jnp.zeros((NPAD, DH), jnp.float32)
    z16 = jnp.zeros((NPAD, DP), jnp.float32)
    ones16 = jnp.ones((C, DP), jnp.float32)
    W2p = jnp.pad(W2, ((0, 0), (0, DP - W2.shape[1])))
    b1r = b1.reshape(1, D_H)
    b2p = jnp.pad(b2, (0, DP - b2.shape[0])).reshape(1, DP)

    degp = _deg(ei3, ones16, z16)
    dinv, xs2 = _tca(degp, xpad)
    s1p = _agg128(xs2, ei3, z64)
    u = _tcu(xs2, dinv, W1)
    ys = _tcb(s1p, u, dinv, W1, b1r, W2p)
    s2p = _agg16(ys, ei3, z16)
    return _tcc(s2p, ys, dinv, b2p)


# speedup vs baseline: 1.0867x; 1.0243x over previous
import functools

import jax
import jax.numpy as jnp
from jax import lax
from jax.experimental import pallas as pl
from jax.experimental.pallas import tpu as pltpu
from jax.experimental.pallas import tpu_sc as plsc

N = 10000
NPAD = 10240
E = 320000
D_IN = 128
DH = 64
D_H = 256
DP = 16

NC = 2
NS = 16
NW = NC * NS
C = 125
CHT = E // (NS * C)
CHW = E // (NW * C)
ROWS_PT = NPAD // NS

_MESH = plsc.VectorSubcoreMesh(
    core_axis_name="c", subcore_axis_name="s", num_cores=NC, num_subcores=NS)


@functools.partial(
    pl.kernel,
    out_type=jax.ShapeDtypeStruct((NC, NPAD, DP), jnp.float32),
    mesh=_MESH,
    compiler_params=pltpu.CompilerParams(use_tc_tiling_on_sc=False),
    scratch_types=[
        pltpu.VMEM((CHW, C), jnp.int32),
        pltpu.VMEM((C, DP), jnp.float32),
        pltpu.VMEM_SHARED((NPAD, DP), jnp.float32),
    ] + [pltpu.SemaphoreType.DMA] * 8,
)
def _deg(ei_hbm, ones_hbm, zero_hbm, out_hbm, di_v, ones_v, acc_sh,
         sem0, sem1, sem2, sem3, sem4, sem5, sem6, sem7):
    c = lax.axis_index("c")
    s = lax.axis_index("s")
    w = c * NS + s
    base = s * ROWS_PT
    pltpu.sync_copy(zero_hbm.at[pl.ds(base, ROWS_PT)],
                    acc_sh.at[pl.ds(base, ROWS_PT)])
    pltpu.sync_copy(ei_hbm.at[1, pl.ds(w * CHW, CHW)], di_v)
    pltpu.sync_copy(ones_hbm, ones_v)
    plsc.subcore_barrier()

    sems = (sem0, sem1, sem2, sem3, sem4, sem5, sem6, sem7)

    def body(i, carry):
        g = i * 8
        ds = [pltpu.async_copy(ones_v, acc_sh.at[di_v.at[g + k]],
                               sems[k], add=True)
              for k in range(8)]
        for d in ds:
            d.wait()
        return carry

    lax.fori_loop(0, CHW // 8, body, 0)
    plsc.subcore_barrier()
    pltpu.sync_copy(acc_sh.at[pl.ds(base, ROWS_PT)],
                    out_hbm.at[c, pl.ds(base, ROWS_PT)])


@functools.partial(
    pl.kernel,
    out_type=jax.ShapeDtypeStruct((NC, NPAD, DH), jnp.float32),
    mesh=_MESH,
    compiler_params=pltpu.CompilerParams(use_tc_tiling_on_sc=False),
    scratch_types=[
        pltpu.VMEM((CHT, C), jnp.int32),
        pltpu.VMEM((CHT, C), jnp.int32),
    ] + [pltpu.VMEM((C, DH), jnp.float32)] * 5
      + [pltpu.VMEM_SHARED((NPAD, DH), jnp.float32)]
      + [pltpu.SemaphoreType.DMA] * 10,
)
def _agg128(tab_hbm, ei_hbm, zero_hbm, out_hbm,
            si_v, di_v, r0, r1, r2, r3, r4, acc_sh,
            g0, g1, g2, g3, g4, s0, s1, s2, s3, s4):
    c = lax.axis_index("c")
    s = lax.axis_index("s")
    base = s * ROWS_PT
    pltpu.sync_copy(zero_hbm.at[pl.ds(base, ROWS_PT)],
                    acc_sh.at[pl.ds(base, ROWS_PT)])
    pltpu.sync_copy(ei_hbm.at[0, pl.ds(s * CHT, CHT)], si_v)
    pltpu.sync_copy(ei_hbm.at[1, pl.ds(s * CHT, CHT)], di_v)
    plsc.subcore_barrier()

    rows = (r0, r1, r2, r3, r4)
    gsems = (g0, g1, g2, g3, g4)
    ssems = (s0, s1, s2, s3, s4)

    def body(i, carry):
        g = i * 5
        gd = [pltpu.async_copy(tab_hbm.at[c].at[si_v.at[g + k]], rows[k],
                               gsems[k])
              for k in range(5)]
        sd = []
        for k in range(5):
            gd[k].wait()
            sd.append(pltpu.async_copy(rows[k], acc_sh.at[di_v.at[g + k]],
                                       ssems[k], add=True))
        for d in sd:
            d.wait()
        return carry

    lax.fori_loop(0, CHT // 5, body, 0)
    plsc.subcore_barrier()
    pltpu.sync_copy(acc_sh.at[pl.ds(base, ROWS_PT)],
                    out_hbm.at[c, pl.ds(base, ROWS_PT)])


@functools.partial(
    pl.kernel,
    out_type=jax.ShapeDtypeStruct((NC, NPAD, DP), jnp.float32),
    mesh=_MESH,
    compiler_params=pltpu.CompilerParams(use_tc_tiling_on_sc=False),
    scratch_types=[
        pltpu.VMEM((CHW, C), jnp.int32),
        pltpu.VMEM((CHW, C), jnp.int32),
    ] + [pltpu.VMEM((C, DP), jnp.float32)] * 8
      + [pltpu.VMEM_SHARED((NPAD, DP), jnp.float32)]
      + [pltpu.SemaphoreType.DMA] * 16,
)
def _agg16(tab_hbm, ei_hbm, zero_hbm, out_hbm,
           si_v, di_v, r0, r1, r2, r3, r4, r5, r6, r7, acc_sh,
           g0, g1, g2, g3, g4, g5, g6, g7,
           s0, s1, s2, s3, s4, s5, s6, s7):
    c = lax.axis_index("c")
    s = lax.axis_index("s")
    w = c * NS + s
    base = s * ROWS_PT
    pltpu.sync_copy(zero_hbm.at[pl.ds(base, ROWS_PT)],
                    acc_sh.at[pl.ds(base, ROWS_PT)])
    pltpu.sync_copy(ei_hbm.at[0, pl.ds(w * CHW, CHW)], si_v)
    pltpu.sync_copy(ei_hbm.at[1, pl.ds(w * CHW, CHW)], di_v)
    plsc.subcore_barrier()

    rows = (r0, r1, r2, r3, r4, r5, r6, r7)
    gsems = (g0, g1, g2, g3, g4, g5, g6, g7)
    ssems = (s0, s1, s2, s3, s4, s5, s6, s7)

    def body(i, carry):
        g = i * 8
        gd = [pltpu.async_copy(tab_hbm.at[si_v.at[g + k]], rows[k], gsems[k])
              for k in range(8)]
        sd = []
        for k in range(8):
            gd[k].wait()
            sd.append(pltpu.async_copy(rows[k], acc_sh.at[di_v.at[g + k]],
                                       ssems[k], add=True))
        for d in sd:
            d.wait()
        return carry

    lax.fori_loop(0, CHW // 8, body, 0)
    plsc.subcore_barrier()
    pltpu.sync_copy(acc_sh.at[pl.ds(base, ROWS_PT)],
                    out_hbm.at[c, pl.ds(base, ROWS_PT)])


def _tca_body(dp_ref, x_ref, dinv_ref, xs_ref):
    deg = dp_ref[0, :, 0:1] + dp_ref[1, :, 0:1]
    dinv = lax.rsqrt(deg + 1.0)
    dinv_ref[...] = dinv
    v = x_ref[...] * dinv
    xs_ref[0] = v[:, :DH]
    xs_ref[1] = v[:, DH:]


def _tca(degp, xpad):
    return pl.pallas_call(
        _tca_body,
        out_shape=[
            jax.ShapeDtypeStruct((NPAD, 1), jnp.float32),
            jax.ShapeDtypeStruct((NC, NPAD, DH), jnp.float32),
        ],
    )(degp, xpad)


RB = 2560


def _tcu_body(xs_ref, dinv_ref, w1_ref, u_ref):
    dxs = jnp.concatenate([xs_ref[0], xs_ref[1]], axis=1) * dinv_ref[...]
    u_ref[...] = jnp.dot(dxs, w1_ref[...], preferred_element_type=jnp.float32)


def _tcu(xs2, dinv, W1):
    return pl.pallas_call(
        _tcu_body,
        grid=(NPAD // RB,),
        in_specs=[
            pl.BlockSpec((NC, RB, DH), lambda i: (0, i, 0)),
            pl.BlockSpec((RB, 1), lambda i: (i, 0)),
            pl.BlockSpec((D_IN, D_H), lambda i: (0, 0)),
        ],
        out_specs=pl.BlockSpec((RB, D_H), lambda i: (i, 0)),
        out_shape=jax.ShapeDtypeStruct((NPAD, D_H), jnp.float32),
    )(xs2, dinv, W1)


def _tcb_body(s1_ref, u_ref, dinv_ref, w1_ref, b1_ref, w2_ref, ys_ref):
    dinv = dinv_ref[...]
    t = jnp.concatenate([s1_ref[0], s1_ref[1]], axis=1) * dinv
    h = jnp.maximum(
        jnp.dot(t, w1_ref[...], preferred_element_type=jnp.float32)
        + u_ref[...] + b1_ref[...],
        0.0)
    y = jnp.dot(h, w2_ref[...], preferred_element_type=jnp.float32)
    ys_ref[...] = y * dinv


def _tcb(s1p, u, dinv, W1, b1, W2p):
    return pl.pallas_call(
        _tcb_body,
        grid=(NPAD // RB,),
        in_specs=[
            pl.BlockSpec((NC, RB, DH), lambda i: (0, i, 0)),
            pl.BlockSpec((RB, D_H), lambda i: (i, 0)),
            pl.BlockSpec((RB, 1), lambda i: (i, 0)),
            pl.BlockSpec((D_IN, D_H), lambda i: (0, 0)),
            pl.BlockSpec((1, D_H), lambda i: (0, 0)),
            pl.BlockSpec((D_H, DP), lambda i: (0, 0)),
        ],
        out_specs=pl.BlockSpec((RB, DP), lambda i: (i, 0)),
        out_shape=jax.ShapeDtypeStruct((NPAD, DP), jnp.float32),
    )(s1p, u, dinv, W1, b1, W2p)


def _tcc_body(s2p_ref, ys_ref, dinv_ref, b2_ref, out_ref):
    v = ((s2p_ref[0] + s2p_ref[1] + ys_ref[...]) * dinv_ref[...]
         + b2_ref[...])
    out_ref[...] = v[:N, 0:2]


def _tcc(s2p, ys, dinv, b2p):
    return pl.pallas_call(
        _tcc_body,
        out_shape=jax.ShapeDtypeStruct((N, 2), jnp.float32),
    )(s2p, ys, dinv, b2p)


def kernel(x, edge_index, W1, b1, W2, b2):
    ei3 = edge_index.reshape(2, NS * CHT, C)
    xpad = jnp.pad(x, ((0, NPAD - N), (0, 0)))
    z64 = jnp.zeros((NPAD, DH), jnp.float32)
    z16 = jnp.zeros((NPAD, DP), jnp.float32)
    ones16 = jnp.ones((C, DP), jnp.float32)
    W2p = jnp.pad(W2, ((0, 0), (0, DP - W2.shape[1])))
    b1r = b1.reshape(1, D_H)
    b2p = jnp.pad(b2, (0, DP - b2.shape[0])).reshape(1, DP)

    degp = _deg(ei3, ones16, z16)
    dinv, xs2 = _tca(degp, xpad)
    s1p = _agg128(xs2, ei3, z64)
    u = _tcu(xs2, dinv, W1)
    ys = _tcb(s1p, u, dinv, W1, b1r, W2p)
    s2p = _agg16(ys, ei3, z16)
    return _tcc(s2p, ys, dinv, b2p)
